# Initial kernel scaffold; baseline (speedup 1.0000x reference)
import jax
import jax.numpy as jnp
from jax.experimental import pallas as pl


def kernel(x, edge_index, u, W1, b1, W2, b2, W3, b3):
    raise NotImplementedError("write your pallas kernel here")


# speedup vs baseline: 9.9252x; 9.9252x over previous
import functools

import jax
import jax.numpy as jnp
from jax import lax
from jax.experimental import pallas as pl
from jax.experimental.pallas import tpu as pltpu
from jax.experimental.pallas import tpu_sc as plsc

N_NODES = 10000
N_EDGES = 320000
D_FEAT = 128
D_HID = 32

_NC = 2
_NS = 16
_NW = _NC * _NS
_E_PER_W = N_EDGES // _NW

_C1 = 1000
_CH1 = _E_PER_W // _C1
_C3 = 2000
_CH3 = _E_PER_W // _C3

_MESH = plsc.VectorSubcoreMesh(core_axis_name="c", subcore_axis_name="s")
_SC_PARAMS = pltpu.CompilerParams(use_tc_tiling_on_sc=False,
                                  needs_layout_passes=False)


def _node_proj_body(x_ref, w1_ref, b1_ref, a_ref, b_ref):
    xb = x_ref[...]
    a_ref[...] = jnp.dot(xb, w1_ref[0:D_FEAT, :],
                         preferred_element_type=jnp.float32)
    b_ref[...] = jnp.dot(xb, w1_ref[D_FEAT:2 * D_FEAT, :],
                         preferred_element_type=jnp.float32) + b1_ref[...]


_NODE_BLK = 2000


def _node_proj(x, w1, b1):
    grid = (N_NODES // _NODE_BLK,)
    return pl.pallas_call(
        _node_proj_body,
        grid=grid,
        in_specs=[
            pl.BlockSpec((_NODE_BLK, D_FEAT), lambda i: (i, 0)),
            pl.BlockSpec((2 * D_FEAT, D_HID), lambda i: (0, 0)),
            pl.BlockSpec((1, D_HID), lambda i: (0, 0)),
        ],
        out_specs=[
            pl.BlockSpec((_NODE_BLK, D_HID), lambda i: (i, 0)),
            pl.BlockSpec((_NODE_BLK, D_HID), lambda i: (i, 0)),
        ],
        out_shape=[
            jax.ShapeDtypeStruct((N_NODES, D_HID), jnp.float32),
            jax.ShapeDtypeStruct((N_NODES, D_HID), jnp.float32),
        ],
    )(x, w1, b1)


@functools.partial(
    pl.kernel,
    out_type=jax.ShapeDtypeStruct((N_EDGES, D_HID), jnp.float32),
    mesh=_MESH,
    scratch_types=[
        pltpu.VMEM((_C1,), jnp.int32),
        pltpu.VMEM((_C1,), jnp.int32),
        pltpu.VMEM((_C1, D_HID), jnp.float32),
        pltpu.VMEM((_C1, D_HID), jnp.float32),
        pltpu.SemaphoreType.DMA,
        pltpu.SemaphoreType.DMA,
    ],
    compiler_params=_SC_PARAMS,
)
def _gather_sum(a_hbm, b_hbm, row_hbm, col_hbm, g_hbm,
                idxr, idxc, bufa, bufb, sema, semb):
    wid = lax.axis_index("s") * _NC + lax.axis_index("c")

    def chunk(t, carry):
        base = wid * _E_PER_W + t * _C1
        pltpu.sync_copy(row_hbm.at[pl.ds(base, _C1)], idxr)
        pltpu.sync_copy(col_hbm.at[pl.ds(base, _C1)], idxc)
        cpa = pltpu.async_copy(a_hbm.at[idxr], bufa, sema)
        cpb = pltpu.async_copy(b_hbm.at[idxc], bufb, semb)
        cpa.wait()
        cpb.wait()

        def addrow(i, c):
            s0 = pl.ds(0, 16)
            s1 = pl.ds(16, 16)
            bufa[i, s0] = bufa[i, s0] + bufb[i, s0]
            bufa[i, s1] = bufa[i, s1] + bufb[i, s1]
            return c

        lax.fori_loop(0, _C1, addrow, 0)
        pltpu.sync_copy(bufa, g_hbm.at[pl.ds(base, _C1)])
        return carry

    lax.fori_loop(0, _CH1, chunk, 0)


_EDGE_BLK = 16000


def _mlp_tail_body(g_ref, w2_ref, b2_ref, w3_ref, b3_ref, out_ref):
    h1 = jnp.maximum(g_ref[...], 0.0)
    h2 = jnp.maximum(
        jnp.dot(h1, w2_ref[...], preferred_element_type=jnp.float32)
        + b2_ref[...], 0.0)
    out_ref[...] = (jnp.dot(h2, w3_ref[...],
                            preferred_element_type=jnp.float32) + b3_ref[...])


def _mlp_tail(g, w2, b2, w3, b3):
    grid = (N_EDGES // _EDGE_BLK,)
    return pl.pallas_call(
        _mlp_tail_body,
        grid=grid,
        in_specs=[
            pl.BlockSpec((_EDGE_BLK, D_HID), lambda i: (i, 0)),
            pl.BlockSpec((D_HID, D_HID), lambda i: (0, 0)),
            pl.BlockSpec((1, D_HID), lambda i: (0, 0)),
            pl.BlockSpec((D_HID, 1), lambda i: (0, 0)),
            pl.BlockSpec((1, 1), lambda i: (0, 0)),
        ],
        out_specs=pl.BlockSpec((_EDGE_BLK, 1), lambda i: (i, 0)),
        out_shape=jax.ShapeDtypeStruct((N_EDGES, 1), jnp.float32),
    )(g, w2, b2, w3, b3)


@functools.partial(
    pl.kernel,
    out_type=jax.ShapeDtypeStruct((_NW, N_NODES), jnp.float32),
    mesh=_MESH,
    scratch_types=[
        pltpu.VMEM((N_NODES,), jnp.float32),
        pltpu.VMEM((N_NODES,), jnp.float32),
        pltpu.VMEM((_C3,), jnp.float32),
        pltpu.VMEM((_C3,), jnp.int32),
        pltpu.VMEM((_C3,), jnp.int32),
    ],
    compiler_params=_SC_PARAMS,
)
def _scatter_u(w_hbm, row_hbm, col_hbm, u_hbm, out_hbm,
               u_v, f_v, w_v, r_v, c_v):
    wid = lax.axis_index("s") * _NC + lax.axis_index("c")
    pltpu.sync_copy(u_hbm, u_v)

    zeros16 = jnp.zeros((16,), jnp.float32)

    def zero(i, c):
        f_v[pl.ds(i * 16, 16)] = zeros16
        return c

    lax.fori_loop(0, N_NODES // 16, zero, 0)

    def chunk(t, carry):
        base = wid * _E_PER_W + t * _C3
        pltpu.sync_copy(w_hbm.at[pl.ds(base, _C3)], w_v)
        pltpu.sync_copy(row_hbm.at[pl.ds(base, _C3)], r_v)
        pltpu.sync_copy(col_hbm.at[pl.ds(base, _C3)], c_v)

        def vec(i, c):
            s = pl.ds(i * 16, 16)
            c16 = c_v[s]
            r16 = r_v[s]
            w16 = w_v[s]
            uc = plsc.load_gather(u_v, [c16])
            plsc.addupdate_scatter(f_v, [r16], w16 * uc)
            return c

        lax.fori_loop(0, _C3 // 16, vec, 0)
        return carry

    lax.fori_loop(0, _CH3, chunk, 0)
    pltpu.sync_copy(f_v, out_hbm.at[wid])


def _reduce_body(p_ref, f_ref):
    f_ref[...] = jnp.sum(p_ref[...], axis=0, keepdims=True)


def _reduce(parts):
    return pl.pallas_call(
        _reduce_body,
        out_shape=jax.ShapeDtypeStruct((1, N_NODES), jnp.float32),
    )(parts)


def kernel(x, edge_index, u, W1, b1, W2, b2, W3, b3):
    row = edge_index[0]
    col = edge_index[1]
    a, b = _node_proj(x, W1, b1.reshape(1, D_HID))
    g = _gather_sum(a, b, row, col)
    w = _mlp_tail(g, W2, b2.reshape(1, D_HID), W3, b3.reshape(1, 1))
    parts = _scatter_u(w.reshape(N_EDGES), row, col, u)
    f = _reduce(parts)
    return f.reshape(N_NODES)


# speedup vs baseline: 21.2768x; 2.1437x over previous
import functools

import jax
import jax.numpy as jnp
from jax import lax
from jax.experimental import pallas as pl
from jax.experimental.pallas import tpu as pltpu
from jax.experimental.pallas import tpu_sc as plsc

N_NODES = 10000
N_EDGES = 320000
D_FEAT = 128
D_HID = 32

_NC = 2
_NS = 16
_NW = _NC * _NS
_E_PER_W = N_EDGES // _NW

_C1 = 1000
_CH1 = _E_PER_W // _C1
_C3 = 2000
_CH3 = _E_PER_W // _C3

_MESH = plsc.VectorSubcoreMesh(core_axis_name="c", subcore_axis_name="s")
_SC_PARAMS = pltpu.CompilerParams(use_tc_tiling_on_sc=False,
                                  needs_layout_passes=False)


_N4 = N_NODES // 4


def _node_proj_body(xg_ref, w1s_ref, w1d_ref, b1t_ref, a_ref, b_ref):
    xb = xg_ref[...]
    a_ref[...] = jnp.dot(xb, w1s_ref[...], preferred_element_type=jnp.float32)
    b_ref[...] = (jnp.dot(xb, w1d_ref[...], preferred_element_type=jnp.float32)
                  + b1t_ref[...])


def _node_proj(xg, w1s_bd, w1d_bd, b1t):
    return pl.pallas_call(
        _node_proj_body,
        out_shape=[
            jax.ShapeDtypeStruct((_N4, 128), jnp.float32),
            jax.ShapeDtypeStruct((_N4, 128), jnp.float32),
        ],
    )(xg, w1s_bd, w1d_bd, b1t)


@functools.partial(
    pl.kernel,
    out_type=jax.ShapeDtypeStruct((N_EDGES * D_HID,), jnp.float32),
    mesh=_MESH,
    scratch_types=[
        pltpu.VMEM((_C1,), jnp.int32),
        pltpu.VMEM((_C1,), jnp.int32),
        pltpu.VMEM((_C1, D_HID), jnp.float32),
        pltpu.VMEM((_C1, D_HID), jnp.float32),
        pltpu.VMEM((_C1 * D_HID,), jnp.float32),
        pltpu.SemaphoreType.DMA,
        pltpu.SemaphoreType.DMA,
    ],
    compiler_params=_SC_PARAMS,
)
def _gather_sum(a_hbm, b_hbm, row_hbm, col_hbm, g_hbm,
                idxr, idxc, bufa, bufb, bufo, sema, semb):
    wid = lax.axis_index("s") * _NC + lax.axis_index("c")

    def chunk(t, carry):
        base = wid * _E_PER_W + t * _C1
        pltpu.sync_copy(row_hbm.at[pl.ds(base, _C1)], idxr)
        pltpu.sync_copy(col_hbm.at[pl.ds(base, _C1)], idxc)
        cpa = pltpu.async_copy(a_hbm.at[idxr], bufa, sema)
        cpb = pltpu.async_copy(b_hbm.at[idxc], bufb, semb)
        cpa.wait()
        cpb.wait()

        def add8(i, c):
            r0 = i * 8
            for dr in range(8):
                for j in (0, 16):
                    s = pl.ds(j, 16)
                    bufo[pl.ds((r0 + dr) * D_HID + j, 16)] = (
                        bufa[r0 + dr, s] + bufb[r0 + dr, s])
            return c

        lax.fori_loop(0, _C1 // 8, add8, 0)
        pltpu.sync_copy(bufo, g_hbm.at[pl.ds(base * D_HID, _C1 * D_HID)])
        return carry

    lax.fori_loop(0, _CH1, chunk, 0)


_EDGE_BLK = 32768
_B4 = _EDGE_BLK // 4


def _mlp_tail_body(g_ref, w2_ref, b2_ref, w3_ref, b3_ref, out_ref):
    g4 = g_ref[...].reshape(_B4, 128)
    h1 = jnp.maximum(g4, 0.0)
    h2 = jnp.maximum(
        jnp.dot(h1, w2_ref[...], preferred_element_type=jnp.float32)
        + b2_ref[...], 0.0)
    w32 = (jnp.dot(h2, w3_ref[...], preferred_element_type=jnp.float32)
           + b3_ref[...])
    out_ref[...] = w32.reshape(_EDGE_BLK * D_HID)


def _mlp_tail(gflat, w2_bd, b2t, w3_rep, b3rep):
    grid = (pl.cdiv(N_EDGES, _EDGE_BLK),)
    return pl.pallas_call(
        _mlp_tail_body,
        grid=grid,
        in_specs=[
            pl.BlockSpec((_EDGE_BLK * D_HID,), lambda i: (i,)),
            pl.BlockSpec((128, 128), lambda i: (0, 0)),
            pl.BlockSpec((1, 128), lambda i: (0, 0)),
            pl.BlockSpec((128, 128), lambda i: (0, 0)),
            pl.BlockSpec((1, 128), lambda i: (0, 0)),
        ],
        out_specs=pl.BlockSpec((_EDGE_BLK * D_HID,), lambda i: (i,)),
        out_shape=jax.ShapeDtypeStruct((N_EDGES * D_HID,), jnp.float32),
    )(gflat, w2_bd, b2t, w3_rep, b3rep)


@functools.partial(
    pl.kernel,
    out_type=jax.ShapeDtypeStruct((_NW, N_NODES), jnp.float32),
    mesh=_MESH,
    scratch_types=[
        pltpu.VMEM((N_NODES,), jnp.float32),
        pltpu.VMEM((N_NODES,), jnp.float32),
        pltpu.VMEM((_C3 * D_HID,), jnp.float32),
        pltpu.VMEM((_C3,), jnp.int32),
        pltpu.VMEM((_C3,), jnp.int32),
    ],
    compiler_params=_SC_PARAMS,
)
def _scatter_u(w_hbm, row_hbm, col_hbm, u_hbm, out_hbm,
               u_v, f_v, w_v, r_v, c_v):
    wid = lax.axis_index("s") * _NC + lax.axis_index("c")
    pltpu.sync_copy(u_hbm, u_v)

    zeros16 = jnp.zeros((16,), jnp.float32)
    iota16 = lax.iota(jnp.int32, 16)

    def zero(i, c):
        f_v[pl.ds(i * 16, 16)] = zeros16
        return c

    lax.fori_loop(0, N_NODES // 16, zero, 0)

    def chunk(t, carry):
        base = wid * _E_PER_W + t * _C3
        pltpu.sync_copy(w_hbm.at[pl.ds(base * D_HID, _C3 * D_HID)], w_v)
        pltpu.sync_copy(row_hbm.at[pl.ds(base, _C3)], r_v)
        pltpu.sync_copy(col_hbm.at[pl.ds(base, _C3)], c_v)

        def vec(i, c):
            s = pl.ds(i * 16, 16)
            c16 = c_v[s]
            r16 = r_v[s]
            w16 = plsc.load_gather(w_v, [(i * 16 + iota16) * D_HID])
            uc = plsc.load_gather(u_v, [c16])
            plsc.addupdate_scatter(f_v, [r16], w16 * uc)
            return c

        lax.fori_loop(0, _C3 // 16, vec, 0)
        return carry

    lax.fori_loop(0, _CH3, chunk, 0)
    pltpu.sync_copy(f_v, out_hbm.at[wid])


def _reduce_body(p_ref, f_ref):
    f_ref[...] = jnp.sum(p_ref[...], axis=0, keepdims=True)


def _reduce(parts):
    return pl.pallas_call(
        _reduce_body,
        out_shape=jax.ShapeDtypeStruct((1, N_NODES), jnp.float32),
    )(parts)


def kernel(x, edge_index, u, W1, b1, W2, b2, W3, b3):
    row = edge_index[0]
    col = edge_index[1]
    eye4 = jnp.eye(4, dtype=jnp.float32)
    w1s_bd = jnp.kron(eye4, W1[:D_FEAT])
    w1d_bd = jnp.kron(eye4, W1[D_FEAT:])
    b1t = jnp.tile(b1, 4).reshape(1, 128)
    w2_bd = jnp.kron(eye4, W2)
    b2t = jnp.tile(b2, 4).reshape(1, 128)
    w3_rep = jnp.kron(eye4, jnp.tile(W3, (1, D_HID)))
    b3rep = jnp.tile(b3, 128).reshape(1, 128)
    xg = x.reshape(_N4, 4 * D_FEAT)
    a4, b4 = _node_proj(xg, w1s_bd, w1d_bd, b1t)
    a = a4.reshape(N_NODES, D_HID)
    b = b4.reshape(N_NODES, D_HID)
    gflat = _gather_sum(a, b, row, col)
    w32 = _mlp_tail(gflat, w2_bd, b2t, w3_rep, b3rep)
    parts = _scatter_u(w32, row, col, u)
    f = _reduce(parts)
    return f.reshape(N_NODES)


# speedup vs baseline: 27.4071x; 1.2881x over previous
import functools

import jax
import jax.numpy as jnp
from jax import lax
from jax.experimental import pallas as pl
from jax.experimental.pallas import tpu as pltpu
from jax.experimental.pallas import tpu_sc as plsc

N_NODES = 10000
N_EDGES = 320000
D_FEAT = 128
D_HID = 32

_NC = 2
_NS = 16
_NW = _NC * _NS
_E_PER_W = N_EDGES // _NW

_C1 = 400
_CH1 = _E_PER_W // _C1
_C3 = 400
_CH3 = _E_PER_W // _C3

_MESH = plsc.VectorSubcoreMesh(core_axis_name="c", subcore_axis_name="s")
_SC_PARAMS = pltpu.CompilerParams(use_tc_tiling_on_sc=False,
                                  needs_layout_passes=False)


_N4 = N_NODES // 4


def _node_proj_body(xg_ref, w1s_ref, w1d_ref, b1t_ref, a_ref, b_ref):
    xb = xg_ref[...]
    a_ref[...] = jnp.dot(xb, w1s_ref[...], preferred_element_type=jnp.float32)
    b_ref[...] = (jnp.dot(xb, w1d_ref[...], preferred_element_type=jnp.float32)
                  + b1t_ref[...])


def _node_proj(xg, w1s_bd, w1d_bd, b1t):
    return pl.pallas_call(
        _node_proj_body,
        out_shape=[
            jax.ShapeDtypeStruct((_N4, 128), jnp.float32),
            jax.ShapeDtypeStruct((_N4, 128), jnp.float32),
        ],
    )(xg, w1s_bd, w1d_bd, b1t)


@functools.partial(
    pl.kernel,
    out_type=jax.ShapeDtypeStruct((N_EDGES * D_HID,), jnp.float32),
    mesh=_MESH,
    scratch_types=[
        pltpu.VMEM((_E_PER_W,), jnp.int32),
        pltpu.VMEM((_E_PER_W,), jnp.int32),
        pltpu.VMEM((_C1, D_HID), jnp.float32),
        pltpu.VMEM((_C1, D_HID), jnp.float32),
        pltpu.VMEM((_C1, D_HID), jnp.float32),
        pltpu.VMEM((_C1, D_HID), jnp.float32),
        pltpu.VMEM((_C1 * D_HID,), jnp.float32),
        pltpu.VMEM((_C1 * D_HID,), jnp.float32),
        pltpu.SemaphoreType.DMA,
        pltpu.SemaphoreType.DMA,
        pltpu.SemaphoreType.DMA,
        pltpu.SemaphoreType.DMA,
        pltpu.SemaphoreType.DMA,
        pltpu.SemaphoreType.DMA,
    ],
    compiler_params=_SC_PARAMS,
)
def _gather_sum(a_hbm, b_hbm, row_hbm, col_hbm, g_hbm,
                idxr, idxc, bufa0, bufa1, bufb0, bufb1, bufo0, bufo1,
                sga0, sga1, sgb0, sgb1, swo0, swo1):
    wid = lax.axis_index("s") * _NC + lax.axis_index("c")
    ebase = wid * _E_PER_W
    pltpu.sync_copy(row_hbm.at[pl.ds(ebase, _E_PER_W)], idxr)
    pltpu.sync_copy(col_hbm.at[pl.ds(ebase, _E_PER_W)], idxc)

    bufa = (bufa0, bufa1)
    bufb = (bufb0, bufb1)
    bufo = (bufo0, bufo1)
    sga = (sga0, sga1)
    sgb = (sgb0, sgb1)
    swo = (swo0, swo1)

    def issue(t):
        k = t % 2
        ca = pltpu.async_copy(a_hbm.at[idxr.at[pl.ds(t * _C1, _C1)]],
                              bufa[k], sga[k])
        cb = pltpu.async_copy(b_hbm.at[idxc.at[pl.ds(t * _C1, _C1)]],
                              bufb[k], sgb[k])
        return ca, cb

    pend_w = [None, None]
    pend_g = issue(0)
    for t in range(_CH1):
        k = t % 2
        pend_g[0].wait()
        pend_g[1].wait()
        if t + 1 < _CH1:
            pend_g = issue(t + 1)
        if pend_w[k] is not None:
            pend_w[k].wait()

        def add8(i, c, k=k):
            r0 = i * 8
            for dr in range(8):
                for j in (0, 16):
                    s = pl.ds(j, 16)
                    bufo[k][pl.ds((r0 + dr) * D_HID + j, 16)] = (
                        bufa[k][r0 + dr, s] + bufb[k][r0 + dr, s])
            return c

        lax.fori_loop(0, _C1 // 8, add8, 0)
        pend_w[k] = pltpu.async_copy(
            bufo[k],
            g_hbm.at[pl.ds((ebase + t * _C1) * D_HID, _C1 * D_HID)],
            swo[k])
    for p in pend_w:
        if p is not None:
            p.wait()


_EDGE_BLK = 32768
_B4 = _EDGE_BLK // 4


def _mlp_tail_body(g_ref, w2_ref, b2_ref, w3_ref, b3_ref, out_ref):
    g4 = g_ref[...].reshape(_B4, 128)
    h1 = jnp.maximum(g4, 0.0)
    h2 = jnp.maximum(
        jnp.dot(h1, w2_ref[...], preferred_element_type=jnp.float32)
        + b2_ref[...], 0.0)
    w32 = (jnp.dot(h2, w3_ref[...], preferred_element_type=jnp.float32)
           + b3_ref[...])
    out_ref[...] = w32.reshape(_EDGE_BLK * D_HID)


def _mlp_tail(gflat, w2_bd, b2t, w3_rep, b3rep):
    grid = (pl.cdiv(N_EDGES, _EDGE_BLK),)
    return pl.pallas_call(
        _mlp_tail_body,
        grid=grid,
        in_specs=[
            pl.BlockSpec((_EDGE_BLK * D_HID,), lambda i: (i,)),
            pl.BlockSpec((128, 128), lambda i: (0, 0)),
            pl.BlockSpec((1, 128), lambda i: (0, 0)),
            pl.BlockSpec((128, 128), lambda i: (0, 0)),
            pl.BlockSpec((1, 128), lambda i: (0, 0)),
        ],
        out_specs=pl.BlockSpec((_EDGE_BLK * D_HID,), lambda i: (i,)),
        out_shape=jax.ShapeDtypeStruct((N_EDGES * D_HID,), jnp.float32),
    )(gflat, w2_bd, b2t, w3_rep, b3rep)


@functools.partial(
    pl.kernel,
    out_type=jax.ShapeDtypeStruct((_NW * N_NODES,), jnp.float32),
    mesh=_MESH,
    scratch_types=[
        pltpu.VMEM((N_NODES,), jnp.float32),
        pltpu.VMEM((N_NODES,), jnp.float32),
        pltpu.VMEM((_E_PER_W,), jnp.int32),
        pltpu.VMEM((_E_PER_W,), jnp.int32),
        pltpu.VMEM((_C3 * D_HID,), jnp.float32),
        pltpu.VMEM((_C3 * D_HID,), jnp.float32),
        pltpu.SemaphoreType.DMA,
        pltpu.SemaphoreType.DMA,
        pltpu.SemaphoreType.DMA,
        pltpu.SemaphoreType.DMA,
        pltpu.SemaphoreType.DMA,
    ],
    compiler_params=_SC_PARAMS,
)
def _scatter_u(w_hbm, row_hbm, col_hbm, u_hbm, out_hbm,
               u_v, f_v, r_v, c_v, wv0, wv1, su, sr, sc, sw0, sw1):
    wid = lax.axis_index("s") * _NC + lax.axis_index("c")
    ebase = wid * _E_PER_W
    cu = pltpu.async_copy(u_hbm, u_v, su)
    cr = pltpu.async_copy(row_hbm.at[pl.ds(ebase, _E_PER_W)], r_v, sr)
    cc = pltpu.async_copy(col_hbm.at[pl.ds(ebase, _E_PER_W)], c_v, sc)

    zeros16 = jnp.zeros((16,), jnp.float32)
    iota16 = lax.iota(jnp.int32, 16)

    def zero(i, c):
        f_v[pl.ds(i * 16, 16)] = zeros16
        return c

    lax.fori_loop(0, N_NODES // 16, zero, 0)
    cu.wait()
    cr.wait()
    cc.wait()

    wv = (wv0, wv1)
    sw = (sw0, sw1)

    def issue(t):
        k = t % 2
        return pltpu.async_copy(
            w_hbm.at[pl.ds((ebase + t * _C3) * D_HID, _C3 * D_HID)],
            wv[k], sw[k])

    pend = issue(0)
    for t in range(_CH3):
        k = t % 2
        pend.wait()
        if t + 1 < _CH3:
            pend = issue(t + 1)

        def vec(i, c, t=t, k=k):
            s = pl.ds(t * _C3 + i * 16, 16)
            c16 = c_v[s]
            r16 = r_v[s]
            w16 = plsc.load_gather(wv[k], [(i * 16 + iota16) * D_HID])
            uc = plsc.load_gather(u_v, [c16])
            plsc.addupdate_scatter(f_v, [r16], w16 * uc)
            return c

        lax.fori_loop(0, _C3 // 16, vec, 0)
    pltpu.sync_copy(f_v, out_hbm.at[pl.ds(wid * N_NODES, N_NODES)])


def _reduce_body(p_ref, f_ref):
    acc = p_ref[pl.ds(0, N_NODES)]
    for widx in range(1, _NW):
        acc = acc + p_ref[pl.ds(widx * N_NODES, N_NODES)]
    f_ref[...] = acc


def _reduce(parts_flat):
    return pl.pallas_call(
        _reduce_body,
        out_shape=jax.ShapeDtypeStruct((N_NODES,), jnp.float32),
    )(parts_flat)


def _split_idx_body(ei_ref, row_ref, col_ref):
    row_ref[...] = ei_ref[0, :]
    col_ref[...] = ei_ref[1, :]


def _split_idx(edge_index):
    return pl.pallas_call(
        _split_idx_body,
        out_shape=[
            jax.ShapeDtypeStruct((N_EDGES,), jnp.int32),
            jax.ShapeDtypeStruct((N_EDGES,), jnp.int32),
        ],
    )(edge_index)


def kernel(x, edge_index, u, W1, b1, W2, b2, W3, b3):
    row, col = _split_idx(edge_index)
    eye4 = jnp.eye(4, dtype=jnp.float32)
    w1s_bd = jnp.kron(eye4, W1[:D_FEAT])
    w1d_bd = jnp.kron(eye4, W1[D_FEAT:])
    b1t = jnp.tile(b1, 4).reshape(1, 128)
    w2_bd = jnp.kron(eye4, W2)
    b2t = jnp.tile(b2, 4).reshape(1, 128)
    w3_rep = jnp.kron(eye4, jnp.tile(W3, (1, D_HID)))
    b3rep = jnp.tile(b3, 128).reshape(1, 128)
    xg = x.reshape(_N4, 4 * D_FEAT)
    a4, b4 = _node_proj(xg, w1s_bd, w1d_bd, b1t)
    a = a4.reshape(N_NODES, D_HID)
    b = b4.reshape(N_NODES, D_HID)
    gflat = _gather_sum(a, b, row, col)
    w32 = _mlp_tail(gflat, w2_bd, b2t, w3_rep, b3rep)
    parts = _scatter_u(w32, row, col, u)
    return _reduce(parts)


# speedup vs baseline: 31.3993x; 1.1457x over previous
import functools

import jax
import jax.numpy as jnp
from jax import lax
from jax.experimental import pallas as pl
from jax.experimental.pallas import tpu as pltpu
from jax.experimental.pallas import tpu_sc as plsc

N_NODES = 10000
N_EDGES = 320000
D_FEAT = 128
D_HID = 32

_NC = 2
_NS = 16
_NW = _NC * _NS
_E_PER_W = N_EDGES // _NW

_C1 = 400
_CH1 = _E_PER_W // _C1
_C3 = 400
_CH3 = _E_PER_W // _C3

_MESH = plsc.VectorSubcoreMesh(core_axis_name="c", subcore_axis_name="s")
_SC_PARAMS = pltpu.CompilerParams(use_tc_tiling_on_sc=False,
                                  needs_layout_passes=False)


_N4 = N_NODES // 4


def _node_proj_body(xg_ref, w1s_ref, w1d_ref, b1t_ref, ei_ref,
                    a_ref, b_ref, row_ref, col_ref):
    xb = xg_ref[...]
    a_ref[...] = jnp.dot(xb, w1s_ref[...], preferred_element_type=jnp.float32)
    b_ref[...] = (jnp.dot(xb, w1d_ref[...], preferred_element_type=jnp.float32)
                  + b1t_ref[...])
    row_ref[...] = ei_ref[0, :]
    col_ref[...] = ei_ref[1, :]


def _node_proj(xg, w1s_bd, w1d_bd, b1t, edge_index):
    return pl.pallas_call(
        _node_proj_body,
        out_shape=[
            jax.ShapeDtypeStruct((_N4, 128), jnp.float32),
            jax.ShapeDtypeStruct((_N4, 128), jnp.float32),
            jax.ShapeDtypeStruct((N_EDGES,), jnp.int32),
            jax.ShapeDtypeStruct((N_EDGES,), jnp.int32),
        ],
    )(xg, w1s_bd, w1d_bd, b1t, edge_index)


_GD = 3


@functools.partial(
    pl.kernel,
    out_type=jax.ShapeDtypeStruct((N_EDGES * D_HID,), jnp.float32),
    mesh=_MESH,
    scratch_types=(
        [pltpu.VMEM((_E_PER_W,), jnp.int32)] * 2
        + [pltpu.VMEM((_C1, D_HID), jnp.float32)] * (2 * _GD)
        + [pltpu.VMEM((_C1 * D_HID,), jnp.float32)] * 2
        + [pltpu.SemaphoreType.DMA] * (2 * _GD + 2)
    ),
    compiler_params=_SC_PARAMS,
)
def _gather_sum(a_hbm, b_hbm, row_hbm, col_hbm, g_hbm, *refs):
    idxr, idxc = refs[0], refs[1]
    bufa = refs[2:2 + _GD]
    bufb = refs[2 + _GD:2 + 2 * _GD]
    bufo = refs[2 + 2 * _GD:4 + 2 * _GD]
    sga = refs[4 + 2 * _GD:4 + 3 * _GD]
    sgb = refs[4 + 3 * _GD:4 + 4 * _GD]
    swo = refs[4 + 4 * _GD:6 + 4 * _GD]
    wid = lax.axis_index("s") * _NC + lax.axis_index("c")
    ebase = wid * _E_PER_W
    pltpu.sync_copy(row_hbm.at[pl.ds(ebase, _E_PER_W)], idxr)
    pltpu.sync_copy(col_hbm.at[pl.ds(ebase, _E_PER_W)], idxc)

    def issue(t):
        k = t % _GD
        ca = pltpu.async_copy(a_hbm.at[idxr.at[pl.ds(t * _C1, _C1)]],
                              bufa[k], sga[k])
        cb = pltpu.async_copy(b_hbm.at[idxc.at[pl.ds(t * _C1, _C1)]],
                              bufb[k], sgb[k])
        return ca, cb

    pend_w = [None, None]
    pend_g = [issue(t) for t in range(min(_GD - 1, _CH1))]
    for t in range(_CH1):
        k = t % _GD
        ko = t % 2
        ca, cb = pend_g.pop(0)
        ca.wait()
        cb.wait()
        if t + _GD - 1 < _CH1:
            pend_g.append(issue(t + _GD - 1))
        if pend_w[ko] is not None:
            pend_w[ko].wait()

        def add8(i, c, k=k, ko=ko):
            r0 = i * 8
            for dr in range(8):
                for j in (0, 16):
                    s = pl.ds(j, 16)
                    bufo[ko][pl.ds((r0 + dr) * D_HID + j, 16)] = (
                        bufa[k][r0 + dr, s] + bufb[k][r0 + dr, s])
            return c

        lax.fori_loop(0, _C1 // 8, add8, 0)
        pend_w[ko] = pltpu.async_copy(
            bufo[ko],
            g_hbm.at[pl.ds((ebase + t * _C1) * D_HID, _C1 * D_HID)],
            swo[ko])
    for p in pend_w:
        if p is not None:
            p.wait()


_EDGE_BLK = 32768
_B4 = _EDGE_BLK // 4


def _mlp_tail_body(g_ref, w2_ref, b2_ref, w3_ref, b3_ref, out_ref):
    g4 = g_ref[...].reshape(_B4, 128)
    h1 = jnp.maximum(g4, 0.0)
    h2 = jnp.maximum(
        jnp.dot(h1, w2_ref[...], preferred_element_type=jnp.float32)
        + b2_ref[...], 0.0)
    w32 = (jnp.dot(h2, w3_ref[...], preferred_element_type=jnp.float32)
           + b3_ref[...])
    out_ref[...] = w32.reshape(_EDGE_BLK * D_HID)


def _mlp_tail(gflat, w2_bd, b2t, w3_rep, b3rep):
    grid = (pl.cdiv(N_EDGES, _EDGE_BLK),)
    return pl.pallas_call(
        _mlp_tail_body,
        grid=grid,
        in_specs=[
            pl.BlockSpec((_EDGE_BLK * D_HID,), lambda i: (i,)),
            pl.BlockSpec((128, 128), lambda i: (0, 0)),
            pl.BlockSpec((1, 128), lambda i: (0, 0)),
            pl.BlockSpec((128, 128), lambda i: (0, 0)),
            pl.BlockSpec((1, 128), lambda i: (0, 0)),
        ],
        out_specs=pl.BlockSpec((_EDGE_BLK * D_HID,), lambda i: (i,)),
        out_shape=jax.ShapeDtypeStruct((N_EDGES * D_HID,), jnp.float32),
    )(gflat, w2_bd, b2t, w3_rep, b3rep)


_SD = 4


@functools.partial(
    pl.kernel,
    out_type=jax.ShapeDtypeStruct((_NW * N_NODES,), jnp.float32),
    mesh=_MESH,
    scratch_types=(
        [pltpu.VMEM((N_NODES,), jnp.float32)] * 2
        + [pltpu.VMEM((_E_PER_W,), jnp.int32)] * 2
        + [pltpu.VMEM((_C3 * D_HID,), jnp.float32)] * _SD
        + [pltpu.SemaphoreType.DMA] * (_SD + 3)
    ),
    compiler_params=_SC_PARAMS,
)
def _scatter_u(w_hbm, row_hbm, col_hbm, u_hbm, out_hbm, *refs):
    u_v, f_v, r_v, c_v = refs[0], refs[1], refs[2], refs[3]
    wv = refs[4:4 + _SD]
    sw = refs[4 + _SD:4 + 2 * _SD]
    su, sr, sc = refs[4 + 2 * _SD:7 + 2 * _SD]
    wid = lax.axis_index("s") * _NC + lax.axis_index("c")
    ebase = wid * _E_PER_W
    cu = pltpu.async_copy(u_hbm, u_v, su)
    cr = pltpu.async_copy(row_hbm.at[pl.ds(ebase, _E_PER_W)], r_v, sr)
    cc = pltpu.async_copy(col_hbm.at[pl.ds(ebase, _E_PER_W)], c_v, sc)

    zeros16 = jnp.zeros((16,), jnp.float32)
    iota16 = lax.iota(jnp.int32, 16)

    def zero(i, c):
        f_v[pl.ds(i * 16, 16)] = zeros16
        return c

    lax.fori_loop(0, N_NODES // 16, zero, 0)
    cu.wait()
    cr.wait()
    cc.wait()

    def issue(t):
        k = t % _SD
        return pltpu.async_copy(
            w_hbm.at[pl.ds((ebase + t * _C3) * D_HID, _C3 * D_HID)],
            wv[k], sw[k])

    pend = [issue(t) for t in range(min(_SD - 1, _CH3))]
    for t in range(_CH3):
        k = t % _SD
        pend.pop(0).wait()
        if t + _SD - 1 < _CH3:
            pend.append(issue(t + _SD - 1))

        def vec(i, c, t=t, k=k):
            s = pl.ds(t * _C3 + i * 16, 16)
            c16 = c_v[s]
            r16 = r_v[s]
            w16 = plsc.load_gather(wv[k], [(i * 16 + iota16) * D_HID])
            uc = plsc.load_gather(u_v, [c16])
            plsc.addupdate_scatter(f_v, [r16], w16 * uc)
            return c

        lax.fori_loop(0, _C3 // 16, vec, 0)
    pltpu.sync_copy(f_v, out_hbm.at[pl.ds(wid * N_NODES, N_NODES)])


def _reduce_body(p_ref, f_ref):
    acc = p_ref[pl.ds(0, N_NODES)]
    for widx in range(1, _NW):
        acc = acc + p_ref[pl.ds(widx * N_NODES, N_NODES)]
    f_ref[...] = acc


def _reduce(parts_flat):
    return pl.pallas_call(
        _reduce_body,
        out_shape=jax.ShapeDtypeStruct((N_NODES,), jnp.float32),
    )(parts_flat)


def kernel(x, edge_index, u, W1, b1, W2, b2, W3, b3):
    eye4 = jnp.eye(4, dtype=jnp.float32)
    w1s_bd = jnp.kron(eye4, W1[:D_FEAT])
    w1d_bd = jnp.kron(eye4, W1[D_FEAT:])
    b1t = jnp.tile(b1, 4).reshape(1, 128)
    w2_bd = jnp.kron(eye4, W2)
    b2t = jnp.tile(b2, 4).reshape(1, 128)
    w3_rep = jnp.kron(eye4, jnp.tile(W3, (1, D_HID)))
    b3rep = jnp.tile(b3, 128).reshape(1, 128)
    xg = x.reshape(_N4, 4 * D_FEAT)
    a4, b4, row, col = _node_proj(xg, w1s_bd, w1d_bd, b1t, edge_index)
    a = a4.reshape(N_NODES, D_HID)
    b = b4.reshape(N_NODES, D_HID)
    gflat = _gather_sum(a, b, row, col)
    w32 = _mlp_tail(gflat, w2_bd, b2t, w3_rep, b3rep)
    parts = _scatter_u(w32, row, col, u)
    return _reduce(parts)
